# baseline (speedup 1.0000x reference)
import functools

import jax
import jax.numpy as jnp
from jax import lax
from jax.experimental import pallas as pl
from jax.experimental.pallas import tpu as pltpu
from jax.experimental.pallas import tpu_sc as plsc

R = 10
_TB = 2048
_B_SC = 2
_CHUNK = 16


def _fused_kernel(len_ref, x_ref, w_ref, o_ref, smax_ref, smin_ref):
    i = pl.program_id(0)
    TB = x_ref.shape[0]
    cpb = 4096 // TB
    b = i // cpb
    c = i % cpb
    L = len_ref[b]

    s = jnp.dot(x_ref[...], w_ref[...],
                preferred_element_type=jnp.float32).T

    t = c * TB + lax.broadcasted_iota(jnp.int32, (1, TB), 1)

    @pl.when(c == 0)
    def _init():
        for r in range(R):
            smax_ref[r] = jnp.float32(-jnp.inf)
            smin_ref[r] = jnp.float32(jnp.inf)

    for r in range(R):
        start = (r * L) // R
        end = ((r + 1) * L + R - 1) // R
        mask = (t >= start) & (t < end)
        cmax = jnp.max(jnp.where(mask, s, -jnp.inf))
        cmin = jnp.min(jnp.where(mask, s, jnp.inf))
        smax_ref[r] = jnp.maximum(smax_ref[r], cmax)
        smin_ref[r] = jnp.minimum(smin_ref[r], cmin)

    @pl.when(c == cpb - 1)
    def _finish():
        acc = jnp.float32(0.0)
        for r in range(R):
            acc = acc + smax_ref[r] + smin_ref[r]
        o_ref[0, 0, :] = jnp.full((128,), jax.nn.sigmoid(acc),
                                  dtype=jnp.float32)


def _tc_part(xf, lengths, W, n_tc, T, D):
    nt = T // _TB
    return pl.pallas_call(
        _fused_kernel,
        grid=(n_tc * nt,),
        in_specs=[
            pl.BlockSpec(memory_space=pltpu.SMEM),
            pl.BlockSpec((_TB, D), lambda i: (i, 0)),
            pl.BlockSpec((D, 1), lambda i: (0, 0)),
        ],
        out_specs=pl.BlockSpec((1, 1, 128), lambda i: (i // nt, 0, 0)),
        out_shape=jax.ShapeDtypeStruct((n_tc, 1, 128), jnp.float32),
        scratch_shapes=[
            pltpu.SMEM((R,), jnp.float32),
            pltpu.SMEM((R,), jnp.float32),
        ],
    )(lengths, xf, W)


def _make_sc_kernel(B, T, D, nsc):
    info = plsc.get_sparse_core_info()
    NC, NS, NL = info.num_cores, info.num_subcores, info.num_lanes
    npc = nsc // NC
    rpw = T // NS
    nch = rpw // _CHUNK
    b0 = B - nsc
    mesh = plsc.VectorSubcoreMesh(core_axis_name="c", subcore_axis_name="s")

    @functools.partial(
        pl.kernel, mesh=mesh,
        out_type=jax.ShapeDtypeStruct((nsc, NL), jnp.float32),
        scratch_types=[
            pltpu.VMEM((D,), jnp.float32),
            pltpu.VMEM((2, _CHUNK, D), jnp.float32),
            pltpu.VMEM((rpw,), jnp.float32),
            pltpu.VMEM((NL,), jnp.int32),
            pltpu.VMEM((NS, NL), jnp.float32),
            pltpu.VMEM((NS, NL), jnp.float32),
            pltpu.VMEM((NL,), jnp.float32),
            pltpu.VMEM_SHARED((npc, NS, NL), jnp.float32),
            pltpu.VMEM_SHARED((npc, NS, NL), jnp.float32),
            pltpu.SemaphoreType.DMA,
            pltpu.SemaphoreType.DMA,
        ],
    )
    def sc_kernel(x_hbm, len_hbm, w_hbm, out_hbm, w_v, xbuf, sc_v, len_v,
                  gmax_v, gmin_v, out_v, shmax, shmin, sem0, sem1):
        cid = lax.axis_index("c")
        sid = lax.axis_index("s")
        lane = lax.iota(jnp.int32, NL)
        def _shuf(v, sh):
            return v.at[lane ^ sh].get(mode="promise_in_bounds")

        def _hsum(v):
            for sh in (8, 4, 2, 1):
                v = v + _shuf(v, sh)
            return v

        def _hmax(v):
            for sh in (8, 4, 2, 1):
                v = jnp.maximum(v, _shuf(v, sh))
            return v

        def _hmin(v):
            for sh in (8, 4, 2, 1):
                v = jnp.minimum(v, _shuf(v, sh))
            return v

        pltpu.sync_copy(w_hbm, w_v)
        pltpu.sync_copy(len_hbm, len_v)
        lenvec = len_v[...]

        sems = (sem0, sem1)

        def dma_start(g):
            k, c = divmod(g, nch)
            bg = k * NC + cid
            row = (b0 + bg) * T + sid * rpw + c * _CHUNK
            return pltpu.async_copy(
                x_hbm.at[pl.ds(row, _CHUNK), :], xbuf.at[g % 2],
                sems[g % 2])

        total = npc * nch
        pending = dma_start(0)
        for g in range(total):
            k, c = divmod(g, nch)
            nxt = dma_start(g + 1) if g + 1 < total else None
            pending.wait()
            buf = g % 2

            def body(j, accs):
                wv = w_v[pl.ds(j * NL, NL)]
                return tuple(
                    acc + xbuf[buf, rr, pl.ds(j * NL, NL)] * wv
                    for rr, acc in enumerate(accs))

            accs = lax.fori_loop(
                0, D // NL, body,
                tuple(jnp.zeros((NL,), jnp.float32) for _ in range(_CHUNK)))
            svec = jnp.zeros((NL,), jnp.float32)
            for rr in range(_CHUNK):
                svec = jnp.where(lane == rr, _hsum(accs[rr]), svec)
            sc_v[pl.ds(c * _CHUNK, _CHUNK)] = svec

            if c == nch - 1:
                bg = k * NC + cid
                Lv = lenvec.at[jnp.full((NL,), b0, jnp.int32) + bg].get(
                    mode="promise_in_bounds")
                smax = jnp.full((NL,), -jnp.inf, jnp.float32)
                smin = jnp.full((NL,), jnp.inf, jnp.float32)
                Lf = Lv.astype(jnp.float32)
                for r in range(R):
                    startv = (r * Lf / R).astype(jnp.int32)
                    endv = (((r + 1) * Lf + (R - 1)) / R).astype(jnp.int32)
                    amax = jnp.full((NL,), -jnp.inf, jnp.float32)
                    amin = jnp.full((NL,), jnp.inf, jnp.float32)
                    for kk in range(rpw // NL):
                        t = sid * rpw + kk * NL + lane
                        m = (t >= startv) & (t < endv)
                        sl = sc_v[pl.ds(kk * NL, NL)]
                        amax = jnp.maximum(amax, jnp.where(m, sl, -jnp.inf))
                        amin = jnp.minimum(amin, jnp.where(m, sl, jnp.inf))
                    smax = jnp.where(lane == r, _hmax(amax), smax)
                    smin = jnp.where(lane == r, _hmin(amin), smin)
                out_v[...] = smax
                pltpu.sync_copy(out_v, shmax.at[k, sid])
                out_v[...] = smin
                pltpu.sync_copy(out_v, shmin.at[k, sid])
            pending = nxt

        plsc.subcore_barrier()

        for k in range(npc):
            @pl.when(sid == k)
            def _merge():
                bg = k * NC + cid
                pltpu.sync_copy(shmax.at[k], gmax_v)
                pltpu.sync_copy(shmin.at[k], gmin_v)
                mx = gmax_v[0, :]
                mn = gmin_v[0, :]
                for w in range(1, NS):
                    mx = jnp.maximum(mx, gmax_v[w, :])
                    mn = jnp.minimum(mn, gmin_v[w, :])
                contrib = jnp.where(lane < R, mx + mn,
                                    jnp.zeros((NL,), jnp.float32))
                accv = _hsum(contrib)
                out_v[...] = 1.0 / (1.0 + jnp.exp(-accv))
                pltpu.sync_copy(out_v, out_hbm.at[bg])

    return sc_kernel


@jax.jit
def kernel(x, lengths, W):
    B, T, D = x.shape
    n_tc = B - _B_SC
    xf = x.reshape(B * T, D)
    tc_out = _tc_part(xf, lengths, W, n_tc, T, D)

    if _B_SC:
        len16 = jnp.concatenate([lengths, jnp.zeros((16 - B,), jnp.int32)])
        sc_out = _make_sc_kernel(B, T, D, _B_SC)(xf, len16, W.reshape(D))
        return jnp.concatenate([tc_out[:, 0, 0], sc_out[:, 0]])
    return tc_out[:, 0, 0]
